# baseline (speedup 1.0000x reference)
import functools

import jax
import jax.numpy as jnp
from jax import lax
from jax.experimental import pallas as pl
from jax.experimental.pallas import tpu as pltpu
from jax.experimental.pallas import tpu_sc as plsc

TOP_K = 8
N_EXP = 64
SCALING = 2.5
N_TOK = 32768
NC, NS, L = 2, 16, 16
NW = NC * NS
UNROLL_T = 2
N_SPLIT = 2

_mesh = plsc.VectorSubcoreMesh(core_axis_name="c", subcore_axis_name="s")


def _make_router(nt):
    tpw = nt // NW
    n_step = tpw // (L * UNROLL_T)

    @functools.partial(
        pl.kernel,
        out_type=(
            jax.ShapeDtypeStruct((nt, TOP_K), jnp.float32),
            jax.ShapeDtypeStruct((nt, TOP_K), jnp.int32),
            jax.ShapeDtypeStruct((NW, N_EXP), jnp.int32),
        ),
        mesh=_mesh,
        compiler_params=pltpu.CompilerParams(
            needs_layout_passes=False, use_tc_tiling_on_sc=False),
        scratch_types=(
            pltpu.VMEM((tpw, N_EXP), jnp.float32),
            pltpu.VMEM((tpw, TOP_K), jnp.float32),
            pltpu.VMEM((tpw, TOP_K), jnp.int32),
            pltpu.VMEM((N_EXP,), jnp.float32),
            pltpu.VMEM((N_EXP, UNROLL_T * L), jnp.int32),
            pltpu.VMEM((N_EXP,), jnp.int32),
        ),
    )
    def _router(logits_hbm, bias_hbm, w_hbm, i_hbm, h_hbm,
                x_v, w_v, i_v, b_v, hist_v, hred_v):
        c = lax.axis_index("c")
        s = lax.axis_index("s")
        wid = s * NC + c
        base = wid * tpw

        iota = jnp.arange(L, dtype=jnp.int32)
        zeros_i = jnp.zeros((L,), jnp.int32)
        ones_i = jnp.ones((L,), jnp.int32)

        pltpu.sync_copy(bias_hbm, b_v)
        pltpu.sync_copy(logits_hbm.at[pl.ds(base, tpw)], x_v)

        def _zero_body(e, _):
            for u in range(UNROLL_T):
                hist_v[e, pl.ds(u * L, L)] = zeros_i
            return 0
        lax.fori_loop(0, N_EXP, _zero_body, 0)

        neg_inf = jnp.full((L,), -jnp.inf, jnp.float32)

        def step(g0, _):
            toks = [(g0 * UNROLL_T + u) * L + iota for u in range(UNROLL_T)]

            def expert_body(e, carry):
                ms, mis = carry
                new_ms = [list(ms[u]) for u in range(UNROLL_T)]
                new_mis = [list(mis[u]) for u in range(UNROLL_T)]
                e_splat = zeros_i + e
                bias_e = plsc.load_gather(b_v, [e_splat])
                for u in range(UNROLL_T):
                    x = plsc.load_gather(x_v, [toks[u], e_splat])
                    v = 1.0 / (1.0 + jnp.exp(-x)) + bias_e
                    vi = e_splat
                    m = new_ms[u]
                    mi = new_mis[u]
                    for j in range(TOP_K):
                        b = v > m[j]
                        m[j], v = jnp.where(b, v, m[j]), jnp.where(b, m[j], v)
                        mi[j], vi = jnp.where(b, vi, mi[j]), jnp.where(b, mi[j], vi)
                return (tuple(tuple(r) for r in new_ms),
                        tuple(tuple(r) for r in new_mis))

            init = (
                tuple(tuple(neg_inf for _ in range(TOP_K)) for _ in range(UNROLL_T)),
                tuple(tuple(zeros_i for _ in range(TOP_K)) for _ in range(UNROLL_T)),
            )
            ms, mis = lax.fori_loop(0, N_EXP, expert_body, init)

            for u in range(UNROLL_T):
                m, mi = ms[u], mis[u]
                sv = [m[j] - plsc.load_gather(b_v, [mi[j]]) for j in range(TOP_K)]
                den = sv[0]
                for j in range(1, TOP_K):
                    den = den + sv[j]
                fac = SCALING / (den + 1e-20)
                for j in range(TOP_K):
                    jcol = zeros_i + j
                    plsc.store_scatter(w_v, [toks[u], jcol], sv[j] * fac)
                    plsc.store_scatter(i_v, [toks[u], jcol], mi[j])
                    plsc.addupdate_scatter(hist_v, [mi[j], iota + u * L], ones_i)
            return 0

        lax.fori_loop(0, n_step, step, 0)

        pltpu.sync_copy(w_v, w_hbm.at[pl.ds(base, tpw)])
        pltpu.sync_copy(i_v, i_hbm.at[pl.ds(base, tpw)])

        for f in range(N_EXP // L):
            rows = iota + L * f
            acc = plsc.load_gather(hist_v, [rows, zeros_i])
            for u in range(1, UNROLL_T * L):
                acc = acc + plsc.load_gather(hist_v, [rows, zeros_i + u])
            hred_v[pl.ds(L * f, L)] = acc
        pltpu.sync_copy(hred_v, h_hbm.at[wid])

    return _router


_router_part = _make_router(N_TOK // N_SPLIT)


def _hist_sum_body(*refs):
    hs = refs[:-1]
    o_ref = refs[-1]
    acc = jnp.sum(hs[0][...], axis=0)
    for h in hs[1:]:
        acc = acc + jnp.sum(h[...], axis=0)
    o_ref[...] = acc


_hist_sum = pl.pallas_call(
    _hist_sum_body,
    out_shape=jax.ShapeDtypeStruct((N_EXP,), jnp.int32),
)


def kernel(logits, e_score_correction_bias):
    part = N_TOK // N_SPLIT
    ws, idxs, hps = [], [], []
    for p in range(N_SPLIT):
        w, i, hp = _router_part(
            lax.slice_in_dim(logits, p * part, (p + 1) * part),
            e_score_correction_bias)
        ws.append(w)
        idxs.append(i)
        hps.append(hp)
    topk_weight = jnp.concatenate(ws, axis=0)
    topk_idx = jnp.concatenate(idxs, axis=0)
    tokens_per_expert = _hist_sum(*hps)
    return (logits, topk_weight, topk_idx, tokens_per_expert)


# speedup vs baseline: 1.0268x; 1.0268x over previous
import functools

import jax
import jax.numpy as jnp
from jax import lax
from jax.experimental import pallas as pl
from jax.experimental.pallas import tpu as pltpu
from jax.experimental.pallas import tpu_sc as plsc

TOP_K = 8
N_EXP = 64
SCALING = 2.5
N_TOK = 32768
NC, NS, L = 2, 16, 16
NW = NC * NS
UNROLL_T = 2
N_SPLIT = 1

_mesh = plsc.VectorSubcoreMesh(core_axis_name="c", subcore_axis_name="s")


def _make_router(nt):
    tpw = nt // NW
    n_step = tpw // (L * UNROLL_T)

    @functools.partial(
        pl.kernel,
        out_type=(
            jax.ShapeDtypeStruct((nt, TOP_K), jnp.float32),
            jax.ShapeDtypeStruct((nt, TOP_K), jnp.int32),
            jax.ShapeDtypeStruct((NW, N_EXP), jnp.int32),
        ),
        mesh=_mesh,
        compiler_params=pltpu.CompilerParams(
            needs_layout_passes=False, use_tc_tiling_on_sc=False),
        scratch_types=(
            pltpu.VMEM((tpw, N_EXP), jnp.float32),
            pltpu.VMEM((tpw, TOP_K), jnp.float32),
            pltpu.VMEM((tpw, TOP_K), jnp.int32),
            pltpu.VMEM((N_EXP,), jnp.float32),
            pltpu.VMEM((N_EXP, UNROLL_T * L), jnp.int32),
            pltpu.VMEM((N_EXP,), jnp.int32),
        ),
    )
    def _router(logits_hbm, bias_hbm, w_hbm, i_hbm, h_hbm,
                x_v, w_v, i_v, b_v, hist_v, hred_v):
        c = lax.axis_index("c")
        s = lax.axis_index("s")
        wid = s * NC + c
        base = wid * tpw

        iota = jnp.arange(L, dtype=jnp.int32)
        zeros_i = jnp.zeros((L,), jnp.int32)
        ones_i = jnp.ones((L,), jnp.int32)

        pltpu.sync_copy(bias_hbm, b_v)
        pltpu.sync_copy(logits_hbm.at[pl.ds(base, tpw)], x_v)

        def _zero_body(e, _):
            for u in range(UNROLL_T):
                hist_v[e, pl.ds(u * L, L)] = zeros_i
            return 0
        lax.fori_loop(0, N_EXP, _zero_body, 0)

        neg_inf = jnp.full((L,), -jnp.inf, jnp.float32)

        def step(g0, _):
            toks = [(g0 * UNROLL_T + u) * L + iota for u in range(UNROLL_T)]

            def expert_body(e, carry):
                ms, mis = carry
                new_ms = [list(ms[u]) for u in range(UNROLL_T)]
                new_mis = [list(mis[u]) for u in range(UNROLL_T)]
                e_splat = zeros_i + e
                bias_e = plsc.load_gather(b_v, [e_splat])
                for u in range(UNROLL_T):
                    x = plsc.load_gather(x_v, [toks[u], e_splat])
                    v = 1.0 / (1.0 + jnp.exp(-x)) + bias_e
                    vi = e_splat
                    m = new_ms[u]
                    mi = new_mis[u]
                    for j in range(TOP_K):
                        b = v > m[j]
                        m[j], v = jnp.where(b, v, m[j]), jnp.where(b, m[j], v)
                        mi[j], vi = jnp.where(b, vi, mi[j]), jnp.where(b, mi[j], vi)
                return (tuple(tuple(r) for r in new_ms),
                        tuple(tuple(r) for r in new_mis))

            init = (
                tuple(tuple(neg_inf for _ in range(TOP_K)) for _ in range(UNROLL_T)),
                tuple(tuple(zeros_i for _ in range(TOP_K)) for _ in range(UNROLL_T)),
            )
            ms, mis = lax.fori_loop(0, N_EXP, expert_body, init)

            for u in range(UNROLL_T):
                m, mi = ms[u], mis[u]
                sv = [m[j] - plsc.load_gather(b_v, [mi[j]]) for j in range(TOP_K)]
                den = sv[0]
                for j in range(1, TOP_K):
                    den = den + sv[j]
                fac = SCALING / (den + 1e-20)
                for j in range(TOP_K):
                    jcol = zeros_i + j
                    plsc.store_scatter(w_v, [toks[u], jcol], sv[j] * fac)
                    plsc.store_scatter(i_v, [toks[u], jcol], mi[j])
                    plsc.addupdate_scatter(hist_v, [mi[j], iota + u * L], ones_i)
            return 0

        lax.fori_loop(0, n_step, step, 0)

        pltpu.sync_copy(w_v, w_hbm.at[pl.ds(base, tpw)])
        pltpu.sync_copy(i_v, i_hbm.at[pl.ds(base, tpw)])

        for f in range(N_EXP // L):
            rows = iota + L * f
            acc = plsc.load_gather(hist_v, [rows, zeros_i])
            for u in range(1, UNROLL_T * L):
                acc = acc + plsc.load_gather(hist_v, [rows, zeros_i + u])
            hred_v[pl.ds(L * f, L)] = acc
        pltpu.sync_copy(hred_v, h_hbm.at[wid])

    return _router


_router_part = _make_router(N_TOK // N_SPLIT)


def _hist_sum_body(*refs):
    hs = refs[:-1]
    o_ref = refs[-1]
    acc = jnp.sum(hs[0][...], axis=0)
    for h in hs[1:]:
        acc = acc + jnp.sum(h[...], axis=0)
    o_ref[...] = acc


_hist_sum = pl.pallas_call(
    _hist_sum_body,
    out_shape=jax.ShapeDtypeStruct((N_EXP,), jnp.int32),
)


def kernel(logits, e_score_correction_bias):
    if N_SPLIT == 1:
        topk_weight, topk_idx, hp = _router_part(logits, e_score_correction_bias)
        tokens_per_expert = _hist_sum(hp)
        return (logits, topk_weight, topk_idx, tokens_per_expert)
    part = N_TOK // N_SPLIT
    ws, idxs, hps = [], [], []
    for p in range(N_SPLIT):
        w, i, hp = _router_part(
            lax.slice_in_dim(logits, p * part, (p + 1) * part),
            e_score_correction_bias)
        ws.append(w)
        idxs.append(i)
        hps.append(hp)
    topk_weight = jnp.concatenate(ws, axis=0)
    topk_idx = jnp.concatenate(idxs, axis=0)
    tokens_per_expert = _hist_sum(*hps)
    return (logits, topk_weight, topk_idx, tokens_per_expert)
